# Initial kernel scaffold; baseline (speedup 1.0000x reference)
import jax
import jax.numpy as jnp
from jax.experimental import pallas as pl


def kernel(news_tokens, embedding_table):
    raise NotImplementedError("write your pallas kernel here")


# speedup vs baseline: 1.4943x; 1.4943x over previous
import functools

import jax
import jax.numpy as jnp
from jax import lax
from jax.experimental import pallas as pl
from jax.experimental.pallas import tpu as pltpu
from jax.experimental.pallas import tpu_sc as plsc

D = 32
CHUNK = 128
NBUF = 10


def _make_gather(B_flat: int):
  info = plsc.get_sparse_core_info()
  nw = info.num_cores * info.num_subcores
  per_w = B_flat // nw
  nchunk = per_w // CHUNK
  ngroup = nchunk // NBUF
  assert per_w * nw == B_flat and nchunk * CHUNK == per_w
  assert ngroup * NBUF == nchunk and ngroup % 2 == 0

  mesh = plsc.VectorSubcoreMesh(core_axis_name="c", subcore_axis_name="s")

  @functools.partial(
      pl.kernel,
      out_type=jax.ShapeDtypeStruct((B_flat, D), jnp.float32),
      mesh=mesh,
      compiler_params=pltpu.CompilerParams(use_tc_tiling_on_sc=False),
      scratch_types=[
          pltpu.VMEM((nchunk, CHUNK), jnp.int32),
          pltpu.VMEM((2 * NBUF, CHUNK, D), jnp.float32),
          pltpu.SemaphoreType.DMA,
          pltpu.SemaphoreType.DMA,
          pltpu.SemaphoreType.DMA,
          pltpu.SemaphoreType.DMA,
      ],
  )
  def gather_kernel(table_hbm, idx_hbm, out_hbm, idx_v, rows_v,
                    gsem0, gsem1, wsem0, wsem1):
    wid = lax.axis_index("s") * info.num_cores + lax.axis_index("c")
    base = wid * per_w
    gsem = (gsem0, gsem1)
    wsem = (wsem0, wsem1)

    pltpu.sync_copy(idx_hbm.at[wid], idx_v)

    def fire_gathers(half, g):
      for b in range(NBUF):
        pltpu.async_copy(
            table_hbm.at[idx_v.at[g * NBUF + b]],
            rows_v.at[half * NBUF + b],
            gsem[half],
        )

    def wait_gathers(half, g):
      for b in range(NBUF):
        pltpu.make_async_copy(
            table_hbm.at[idx_v.at[g * NBUF + b]],
            rows_v.at[half * NBUF + b],
            gsem[half],
        ).wait()

    def fire_writes(half, g):
      for b in range(NBUF):
        pltpu.async_copy(
            rows_v.at[half * NBUF + b],
            out_hbm.at[pl.ds(base + (g * NBUF + b) * CHUNK, CHUNK)],
            wsem[half],
        )

    def wait_writes(half, g):
      for b in range(NBUF):
        pltpu.make_async_copy(
            rows_v.at[half * NBUF + b],
            out_hbm.at[pl.ds(base + (g * NBUF + b) * CHUNK, CHUNK)],
            wsem[half],
        ).wait()

    fire_gathers(0, 0)

    @pl.loop(0, ngroup, step=2)
    def body(g):
      wait_gathers(0, g)
      fire_writes(0, g)

      @pl.when(g > 0)
      def _():
        wait_writes(1, g - 1)

      fire_gathers(1, g + 1)

      wait_gathers(1, g + 1)
      fire_writes(1, g + 1)

      @pl.when(g + 2 < ngroup)
      def _():
        wait_writes(0, g)
        fire_gathers(0, g + 2)

    wait_writes(0, ngroup - 2)
    wait_writes(1, ngroup - 1)

  return gather_kernel


def kernel(news_tokens, embedding_table):
  batch, seq = news_tokens.shape
  b_flat = batch * seq
  info = plsc.get_sparse_core_info()
  nw = info.num_cores * info.num_subcores
  per_w = b_flat // nw
  idx = news_tokens.astype(jnp.int32).reshape(nw, per_w // CHUNK, CHUNK)
  out = _make_gather(b_flat)(embedding_table, idx)
  return out.reshape(batch, seq, D)
